# Initial kernel scaffold; baseline (speedup 1.0000x reference)
import jax
import jax.numpy as jnp
from jax.experimental import pallas as pl


def kernel(x, edge_index, emb_W1, emb_b1, emb_bn_g, emb_bn_b, emb_W2, emb_b2, gat_W, attn_l, attn_r, gat_bias, bn1_g, bn1_b, ff_W1, ff_b1, ff_W2, ff_b2, bn2_g, bn2_b, dec_W1, dec_b1, dec_bn_g, dec_bn_b, dec_W2, dec_b2):
    raise NotImplementedError("write your pallas kernel here")


# speedup vs baseline: 1.0847x; 1.0847x over previous
import functools

import jax
import jax.numpy as jnp
from jax import lax
from jax.experimental import pallas as pl
from jax.experimental.pallas import tpu as pltpu

N = 10000
E = 320000
DIN = 128
D = 256
H = 16
DH = 16
FF = 512
L = 4
DOUT = 128


def _stats(z):
    m = jnp.mean(z, axis=0)
    v = jnp.mean((z - m) ** 2, axis=0)
    return m, v


def _bnorm(z, m, v, g, b):
    return (z - m) / jnp.sqrt(v + 1e-5) * g + b


def _embed_body(x_ref, w1_ref, b1_ref, g_ref, bb_ref, w2_ref, b2_ref, h_ref):
    z = jnp.dot(x_ref[...], w1_ref[...], preferred_element_type=jnp.float32) + b1_ref[...]
    m, v = _stats(z)
    h1 = jnp.maximum(_bnorm(z, m, v, g_ref[...], bb_ref[...]), 0.0)
    h_ref[...] = jnp.dot(h1, w2_ref[...], preferred_element_type=jnp.float32) + b2_ref[...] + h1


def _attn_body(h_ref, w_ref, a_ref, b_ref, hh_ref, el_ref, er_ref):
    hh = jnp.dot(h_ref[...], w_ref[...], preferred_element_type=jnp.float32)
    hh_ref[...] = hh
    el_ref[...] = jnp.dot(hh, a_ref[...], preferred_element_type=jnp.float32)
    er_ref[...] = jnp.dot(hh, b_ref[...], preferred_element_type=jnp.float32)


def _post1_body(h_ref, rstu_ref, den_ref, e16_ref, gb_ref, g1_ref, b1_ref, out_ref):
    rden = 1.0 / jnp.maximum(den_ref[...], 1e-30)
    scale = jnp.dot(rden, e16_ref[...], preferred_element_type=jnp.float32)
    g = h_ref[...] + rstu_ref[...] * scale + gb_ref[...]
    m, v = _stats(g)
    out_ref[...] = _bnorm(g, m, v, g1_ref[...], b1_ref[...])


def _post2_body(t_ref, fw1_ref, fb1_ref, fw2_ref, fb2_ref, g2_ref, b2_ref, out_ref):
    t = t_ref[...]
    y = jnp.maximum(
        jnp.dot(t, fw1_ref[...], preferred_element_type=jnp.float32) + fb1_ref[...], 0.0)
    y = jnp.dot(y, fw2_ref[...], preferred_element_type=jnp.float32) + fb2_ref[...]
    t2 = t + y
    m2, v2 = _stats(t2)
    out_ref[...] = _bnorm(t2, m2, v2, g2_ref[...], b2_ref[...])


def _dec_body(h_ref, w1_ref, b1_ref, g_ref, bb_ref, w2_ref, b2_ref, out_ref):
    z = jnp.dot(h_ref[...], w1_ref[...], preferred_element_type=jnp.float32) + b1_ref[...]
    m, v = _stats(z)
    d1 = jnp.maximum(_bnorm(z, m, v, g_ref[...], bb_ref[...]), 0.0)
    out_ref[...] = jnp.dot(d1, w2_ref[...], preferred_element_type=jnp.float32) + b2_ref[...]


def _f32(shape):
    return jax.ShapeDtypeStruct(shape, jnp.float32)


def kernel(x, edge_index, emb_W1, emb_b1, emb_bn_g, emb_bn_b, emb_W2, emb_b2,
           gat_W, attn_l, attn_r, gat_bias, bn1_g, bn1_b, ff_W1, ff_b1,
           ff_W2, ff_b2, bn2_g, bn2_b, dec_W1, dec_b1, dec_bn_g, dec_bn_b,
           dec_W2, dec_b2):
    src = edge_index[0]
    dst = edge_index[1]
    r = lambda a: a.reshape(1, -1)

    eyeh = jnp.eye(H, dtype=jnp.float32)
    abig = (attn_l[:, :, :, None] * eyeh[None, :, None, :]).reshape(L, D, H)
    bbig = (attn_r[:, :, :, None] * eyeh[None, :, None, :]).reshape(L, D, H)
    e16 = jnp.kron(eyeh, jnp.ones((1, DH), jnp.float32))

    h = pl.pallas_call(_embed_body, out_shape=_f32((N, D)))(
        x, emb_W1, r(emb_b1), r(emb_bn_g), r(emb_bn_b), emb_W2, r(emb_b2))

    for l in range(L):
        hh, el, er = pl.pallas_call(
            _attn_body, out_shape=[_f32((N, D)), _f32((N, H)), _f32((N, H))],
        )(h, gat_W[l], abig[l], bbig[l])

        e = el[src] + er[dst]
        w = jnp.exp(jnp.where(e > 0, e, 0.2 * e))
        denom = jax.ops.segment_sum(w, dst, num_segments=N)
        msg = w[:, :, None] * hh[src].reshape(E, H, DH)
        rstu = jax.ops.segment_sum(msg, dst, num_segments=N).reshape(N, D)

        t = pl.pallas_call(
            _post1_body, out_shape=_f32((N, D)),
        )(h, rstu, denom, e16, r(gat_bias[l]), r(bn1_g[l]), r(bn1_b[l]))
        h = pl.pallas_call(
            _post2_body, out_shape=_f32((N, D)),
        )(t, ff_W1[l], r(ff_b1[l]), ff_W2[l], r(ff_b2[l]), r(bn2_g[l]), r(bn2_b[l]))

    out = pl.pallas_call(_dec_body, out_shape=_f32((N, DOUT)))(
        h, dec_W1, r(dec_b1), r(dec_bn_g), r(dec_bn_b), dec_W2, r(dec_b2))
    return out


# speedup vs baseline: 39.9178x; 36.7998x over previous
import functools

import jax
import jax.numpy as jnp
from jax import lax
from jax.experimental import pallas as pl
from jax.experimental.pallas import tpu as pltpu
from jax.experimental.pallas import tpu_sc as plsc

N = 10000
E = 320000
DIN = 128
D = 256
H = 16
DH = 16
FF = 512
L = 4
DOUT = 128


def _stats(z):
    m = jnp.mean(z, axis=0)
    v = jnp.mean((z - m) ** 2, axis=0)
    return m, v


def _bnorm(z, m, v, g, b):
    return (z - m) / jnp.sqrt(v + 1e-5) * g + b


def _embed_body(x_ref, w1_ref, b1_ref, g_ref, bb_ref, w2_ref, b2_ref, h_ref):
    z = jnp.dot(x_ref[...], w1_ref[...], preferred_element_type=jnp.float32) + b1_ref[...]
    m, v = _stats(z)
    h1 = jnp.maximum(_bnorm(z, m, v, g_ref[...], bb_ref[...]), 0.0)
    h_ref[...] = jnp.dot(h1, w2_ref[...], preferred_element_type=jnp.float32) + b2_ref[...] + h1


def _attn_body(h_ref, w_ref, a_ref, b_ref, hh_ref, el_ref, er_ref):
    hh = jnp.dot(h_ref[...], w_ref[...], preferred_element_type=jnp.float32)
    hh_ref[0] = hh[:, :128]
    hh_ref[1] = hh[:, 128:]
    el_ref[...] = jnp.dot(hh, a_ref[...], preferred_element_type=jnp.float32)
    er_ref[...] = jnp.dot(hh, b_ref[...], preferred_element_type=jnp.float32)


def _post1_body(h_ref, r0_ref, r1_ref, den_ref, e16_ref, gb_ref, g1_ref, b1_ref, out_ref):
    rstu = jnp.concatenate([r0_ref[...], r1_ref[...]], axis=1)
    rden = 1.0 / jnp.maximum(den_ref[...], 1e-30)
    scale = jnp.dot(rden, e16_ref[...], preferred_element_type=jnp.float32)
    g = h_ref[...] + rstu * scale + gb_ref[...]
    m, v = _stats(g)
    out_ref[...] = _bnorm(g, m, v, g1_ref[...], b1_ref[...])


def _post2_body(t_ref, fw1_ref, fb1_ref, fw2_ref, fb2_ref, g2_ref, b2_ref, out_ref):
    t = t_ref[...]
    y = jnp.maximum(
        jnp.dot(t, fw1_ref[...], preferred_element_type=jnp.float32) + fb1_ref[...], 0.0)
    y = jnp.dot(y, fw2_ref[...], preferred_element_type=jnp.float32) + fb2_ref[...]
    t2 = t + y
    m2, v2 = _stats(t2)
    out_ref[...] = _bnorm(t2, m2, v2, g2_ref[...], b2_ref[...])


def _dec_body(h_ref, w1_ref, b1_ref, g_ref, bb_ref, w2_ref, b2_ref, out_ref):
    z = jnp.dot(h_ref[...], w1_ref[...], preferred_element_type=jnp.float32) + b1_ref[...]
    m, v = _stats(z)
    d1 = jnp.maximum(_bnorm(z, m, v, g_ref[...], bb_ref[...]), 0.0)
    out_ref[...] = jnp.dot(d1, w2_ref[...], preferred_element_type=jnp.float32) + b2_ref[...]


def _f32(shape):
    return jax.ShapeDtypeStruct(shape, jnp.float32)


CHUNK = 128
NTILE = 16
NPAD = 10112
CPT = 157
EPAD = NTILE * CPT * CHUNK
RPT = NPAD // NTILE


def _edge_body(src_h, dst_h, hh_h, el_h, er_h, rstu_h, den_h,
               acc, accd, srcv, dstv, idxv, elb, erb, wb, hhb,
               sem1, sem2, sem3):
    c = lax.axis_index("c")
    s = lax.axis_index("s")

    def zrow(i, carry):
        for k in range(8):
            hhb[i, pl.ds(k * 16, 16)] = jnp.zeros((16,), jnp.float32)
        wb[i] = jnp.zeros((16,), jnp.float32)
        return carry

    lax.fori_loop(0, 128, zrow, 0)
    rbase = pl.multiple_of(s * RPT, 8)
    off = 0
    for sz in (128, 128, 128, 128, RPT - 4 * 128):
        pltpu.sync_copy(hhb.at[pl.ds(0, sz)], acc.at[pl.ds(rbase + off, sz)])
        pltpu.sync_copy(wb.at[pl.ds(0, sz)], accd.at[pl.ds(rbase + off, sz)])
        off += sz
    plsc.subcore_barrier()

    def chunk_body(ci, carry):
        base_e = pl.multiple_of((s * CPT + ci) * CHUNK, 8)
        pltpu.sync_copy(src_h.at[pl.ds(base_e, CHUNK)], srcv)
        pltpu.sync_copy(dst_h.at[pl.ds(base_e, CHUNK)], dstv)

        def addoff(t, cy):
            sl = pl.ds(t * 16, 16)
            idxv[sl] = srcv[sl] + c * N
            return cy

        lax.fori_loop(0, CHUNK // 16, addoff, 0, unroll=True)
        cp1 = pltpu.async_copy(hh_h.at[idxv], hhb, sem1)
        cp2 = pltpu.async_copy(el_h.at[srcv], elb, sem2)
        cp3 = pltpu.async_copy(er_h.at[dstv], erb, sem3)
        cp2.wait()
        cp3.wait()
        cp1.wait()

        def ebody(j, cy):
            ev = elb[j] + erb[j]
            ev = jnp.where(ev > 0, ev, 0.2 * ev)
            wv = jnp.exp(ev)
            wb[j] = wv
            for hloc in range(8):
                lane = jnp.full((16, 1), c * 8 + hloc, jnp.int32)
                wx = lax.gather(
                    wv, lane,
                    lax.GatherDimensionNumbers(
                        offset_dims=(), collapsed_slice_dims=(0,),
                        start_index_map=(0,)),
                    slice_sizes=(1,),
                    mode=lax.GatherScatterMode.PROMISE_IN_BOUNDS)
                col = pl.ds(hloc * 16, 16)
                hhb[j, col] = hhb[j, col] * wx
            return cy

        lax.fori_loop(0, CHUNK, ebody, 0)
        pltpu.sync_copy(hhb, acc.at[dstv], add=True)
        pltpu.sync_copy(wb, accd.at[dstv], add=True)
        return carry

    lax.fori_loop(0, CPT, chunk_body, 0)
    plsc.subcore_barrier()
    pltpu.sync_copy(acc.at[pl.ds(rbase, RPT)], rstu_h.at[c, pl.ds(rbase, RPT)])
    pltpu.sync_copy(accd.at[pl.ds(rbase, RPT)], den_h.at[c, pl.ds(rbase, RPT)])


@functools.lru_cache(maxsize=1)
def _make_edge_pass():
    return functools.partial(
        pl.kernel,
        mesh=plsc.VectorSubcoreMesh(core_axis_name="c", subcore_axis_name="s"),
        compiler_params=pltpu.CompilerParams(use_tc_tiling_on_sc=False),
        out_type=(jax.ShapeDtypeStruct((2, NPAD, 128), jnp.float32),
                  jax.ShapeDtypeStruct((2, NPAD, H), jnp.float32)),
        scratch_types=[
            pltpu.VMEM_SHARED((NPAD, 128), jnp.float32),
            pltpu.VMEM_SHARED((NPAD, H), jnp.float32),
            pltpu.VMEM((CHUNK,), jnp.int32),
            pltpu.VMEM((CHUNK,), jnp.int32),
            pltpu.VMEM((CHUNK,), jnp.int32),
            pltpu.VMEM((CHUNK, H), jnp.float32),
            pltpu.VMEM((CHUNK, H), jnp.float32),
            pltpu.VMEM((CHUNK, H), jnp.float32),
            pltpu.VMEM((CHUNK, 128), jnp.float32),
            pltpu.SemaphoreType.DMA,
            pltpu.SemaphoreType.DMA,
            pltpu.SemaphoreType.DMA,
        ],
    )(_edge_body)


def _edge_pass(*args):
    return _make_edge_pass()(*args)


def kernel(x, edge_index, emb_W1, emb_b1, emb_bn_g, emb_bn_b, emb_W2, emb_b2,
           gat_W, attn_l, attn_r, gat_bias, bn1_g, bn1_b, ff_W1, ff_b1,
           ff_W2, ff_b2, bn2_g, bn2_b, dec_W1, dec_b1, dec_bn_g, dec_bn_b,
           dec_W2, dec_b2):
    src = edge_index[0]
    dst = edge_index[1]
    npad_e = EPAD - E
    src_p = jnp.concatenate([src, jnp.zeros((npad_e,), jnp.int32)])
    dst_p = jnp.concatenate([dst, jnp.full((npad_e,), N, jnp.int32)])
    r = lambda a: a.reshape(1, -1)

    eyeh = jnp.eye(H, dtype=jnp.float32)
    abig = (attn_l[:, :, :, None] * eyeh[None, :, None, :]).reshape(L, D, H)
    bbig = (attn_r[:, :, :, None] * eyeh[None, :, None, :]).reshape(L, D, H)
    e16 = jnp.kron(eyeh, jnp.ones((1, DH), jnp.float32))

    h = pl.pallas_call(_embed_body, out_shape=_f32((N, D)))(
        x, emb_W1, r(emb_b1), r(emb_bn_g), r(emb_bn_b), emb_W2, r(emb_b2))

    for l in range(L):
        hh2, el, er = pl.pallas_call(
            _attn_body, out_shape=[_f32((2, N, 128)), _f32((N, H)), _f32((N, H))],
        )(h, gat_W[l], abig[l], bbig[l])

        hh_cat = hh2.reshape(2 * N, 128)
        el_p = jnp.pad(el, ((0, NPAD - N), (0, 0)))
        er_p = jnp.pad(er, ((0, NPAD - N), (0, 0)))
        rstu2, den2 = _edge_pass(src_p, dst_p, hh_cat, el_p, er_p)

        t = pl.pallas_call(
            _post1_body, out_shape=_f32((N, D)),
        )(h, rstu2[0, :N], rstu2[1, :N], den2[0, :N], e16,
          r(gat_bias[l]), r(bn1_g[l]), r(bn1_b[l]))
        h = pl.pallas_call(
            _post2_body, out_shape=_f32((N, D)),
        )(t, ff_W1[l], r(ff_b1[l]), ff_W2[l], r(ff_b2[l]), r(bn2_g[l]), r(bn2_b[l]))

    out = pl.pallas_call(_dec_body, out_shape=_f32((N, DOUT)))(
        h, dec_W1, r(dec_b1), r(dec_bn_g), r(dec_bn_b), dec_W2, r(dec_b2))
    return out


# speedup vs baseline: 42.6352x; 1.0681x over previous
import functools

import jax
import jax.numpy as jnp
from jax import lax
from jax.experimental import pallas as pl
from jax.experimental.pallas import tpu as pltpu
from jax.experimental.pallas import tpu_sc as plsc

N = 10000
E = 320000
DIN = 128
D = 256
H = 16
DH = 16
FF = 512
L = 4
DOUT = 128


def _stats(z):
    m = jnp.mean(z, axis=0)
    v = jnp.mean((z - m) ** 2, axis=0)
    return m, v


def _bnorm(z, m, v, g, b):
    return (z - m) / jnp.sqrt(v + 1e-5) * g + b


def _embed_body(x_ref, w1_ref, b1_ref, g_ref, bb_ref, w2_ref, b2_ref, h_ref):
    z = jnp.dot(x_ref[...], w1_ref[...], preferred_element_type=jnp.float32) + b1_ref[...]
    m, v = _stats(z)
    h1 = jnp.maximum(_bnorm(z, m, v, g_ref[...], bb_ref[...]), 0.0)
    h_ref[...] = jnp.dot(h1, w2_ref[...], preferred_element_type=jnp.float32) + b2_ref[...] + h1


def _attn_body(h_ref, w_ref, a_ref, b_ref, hhx_ref, er_ref):
    hh = jnp.dot(h_ref[...], w_ref[...], preferred_element_type=jnp.float32)
    el = jnp.dot(hh, a_ref[...], preferred_element_type=jnp.float32)
    hhx_ref[0] = jnp.concatenate([hh[:, :128], el], axis=1)
    hhx_ref[1] = jnp.concatenate([hh[:, 128:], el], axis=1)
    er_ref[...] = jnp.dot(hh, b_ref[...], preferred_element_type=jnp.float32)


def _post1_body(h_ref, r0_ref, r1_ref, den_ref, e16_ref, gb_ref, g1_ref, b1_ref, out_ref):
    rstu = jnp.concatenate([r0_ref[...], r1_ref[...]], axis=1)
    rden = 1.0 / jnp.maximum(den_ref[...], 1e-30)
    scale = jnp.dot(rden, e16_ref[...], preferred_element_type=jnp.float32)
    g = h_ref[...] + rstu * scale + gb_ref[...]
    m, v = _stats(g)
    out_ref[...] = _bnorm(g, m, v, g1_ref[...], b1_ref[...])


def _post2_body(t_ref, fw1_ref, fb1_ref, fw2_ref, fb2_ref, g2_ref, b2_ref, out_ref):
    t = t_ref[...]
    y = jnp.maximum(
        jnp.dot(t, fw1_ref[...], preferred_element_type=jnp.float32) + fb1_ref[...], 0.0)
    y = jnp.dot(y, fw2_ref[...], preferred_element_type=jnp.float32) + fb2_ref[...]
    t2 = t + y
    m2, v2 = _stats(t2)
    out_ref[...] = _bnorm(t2, m2, v2, g2_ref[...], b2_ref[...])


def _dec_body(h_ref, w1_ref, b1_ref, g_ref, bb_ref, w2_ref, b2_ref, out_ref):
    z = jnp.dot(h_ref[...], w1_ref[...], preferred_element_type=jnp.float32) + b1_ref[...]
    m, v = _stats(z)
    d1 = jnp.maximum(_bnorm(z, m, v, g_ref[...], bb_ref[...]), 0.0)
    out_ref[...] = jnp.dot(d1, w2_ref[...], preferred_element_type=jnp.float32) + b2_ref[...]


def _f32(shape):
    return jax.ShapeDtypeStruct(shape, jnp.float32)


CHUNK = 96
NTILE = 16
NPAD = 10112
CPT = 216
EPAD = NTILE * CPT * CHUNK
CROWS = EPAD // CHUNK
RPT = NPAD // NTILE
DW = 144


def _edge_body(src_h, dst_h, hhx_h, er_h, rstu_h,
               acc, sbuf, dbuf, ibuf, erb, hhb, gsem, isem, scsem):
    c = lax.axis_index("c")
    s = lax.axis_index("s")

    def zrow(i, carry):
        for k in range(DW // 16):
            hhb[0, i, pl.ds(k * 16, 16)] = jnp.zeros((16,), jnp.float32)
        return carry

    lax.fori_loop(0, CHUNK, zrow, 0)
    rbase = pl.multiple_of(s * RPT, 8)
    off = 0
    for sz in (96, 96, 96, 96, 96, 96, RPT - 6 * 96):
        pltpu.sync_copy(hhb.at[0, pl.ds(0, sz)], acc.at[pl.ds(rbase + off, sz)])
        off += sz
    plsc.subcore_barrier()

    def idx_copy(chunk_id, i3):
        row = s * CPT + chunk_id
        pltpu.async_copy(src_h.at[row], sbuf.at[i3], isem.at[i3])
        pltpu.async_copy(dst_h.at[row], dbuf.at[i3], isem.at[i3])

    def idx_wait(i3):
        pltpu.make_async_copy(src_h.at[0], sbuf.at[i3], isem.at[i3]).wait()
        pltpu.make_async_copy(dst_h.at[0], dbuf.at[i3], isem.at[i3]).wait()

    def ibuf_compute(i3):
        for t in range(CHUNK // 16):
            sl = pl.ds(t * 16, 16)
            ibuf[i3, sl] = sbuf[i3, sl] + c * N

    def gathers_issue(s2, i3):
        pltpu.async_copy(hhx_h.at[ibuf.at[i3]], hhb.at[s2], gsem.at[s2])
        pltpu.async_copy(er_h.at[dbuf.at[i3]], erb.at[s2], gsem.at[s2])

    def gathers_wait(s2, i3):
        pltpu.make_async_copy(hhx_h.at[ibuf.at[i3]], hhb.at[s2], gsem.at[s2]).wait()
        pltpu.make_async_copy(er_h.at[dbuf.at[i3]], erb.at[s2], gsem.at[s2]).wait()

    def scatter_issue(s2, i3):
        pltpu.async_copy(hhb.at[s2], acc.at[dbuf.at[i3]], scsem.at[s2], add=True)

    def scatter_wait(s2, i3):
        pltpu.make_async_copy(hhb.at[s2], acc.at[dbuf.at[i3]], scsem.at[s2]).wait()

    def compute_chunk(s2):
        def ebody(j, cy):
            elv = hhb[s2, j, pl.ds(128, 16)]
            erv = erb[s2, j]
            ev = elv + erv
            ev = jnp.where(ev > 0, ev, 0.2 * ev)
            wv = jnp.exp(ev)
            hhb[s2, j, pl.ds(128, 16)] = wv
            for hloc in range(8):
                lane = jnp.full((16, 1), c * 8 + hloc, jnp.int32)
                wx = lax.gather(
                    wv, lane,
                    lax.GatherDimensionNumbers(
                        offset_dims=(), collapsed_slice_dims=(0,),
                        start_index_map=(0,)),
                    slice_sizes=(1,),
                    mode=lax.GatherScatterMode.PROMISE_IN_BOUNDS)
                col = pl.ds(hloc * 16, 16)
                hhb[s2, j, col] = hhb[s2, j, col] * wx
            return cy

        lax.fori_loop(0, CHUNK, ebody, 0)

    row0 = s * CPT
    pltpu.sync_copy(src_h.at[row0], sbuf.at[0])
    pltpu.sync_copy(dst_h.at[row0], dbuf.at[0])
    ibuf_compute(0)
    gathers_issue(0, 0)
    idx_copy(1, 1)

    def chunk_iter(ci, carry):
        s0 = lax.rem(ci, 2)
        s1 = 1 - s0
        i0 = lax.rem(ci, 3)
        i1 = lax.rem(ci + 1, 3)
        i2 = lax.rem(ci + 2, 3)

        @pl.when(ci + 1 < CPT)
        def _():
            idx_wait(i1)
            ibuf_compute(i1)

            @pl.when(ci >= 1)
            def _():
                scatter_wait(s1, i2)

            gathers_issue(s1, i1)

        @pl.when(ci + 2 < CPT)
        def _():
            idx_copy(ci + 2, i2)

        gathers_wait(s0, i0)
        compute_chunk(s0)
        scatter_issue(s0, i0)
        return carry

    lax.fori_loop(0, CPT, chunk_iter, 0)
    scatter_wait((CPT - 2) % 2, (CPT - 2) % 3)
    scatter_wait((CPT - 1) % 2, (CPT - 1) % 3)
    plsc.subcore_barrier()
    pltpu.sync_copy(acc.at[pl.ds(rbase, RPT)], rstu_h.at[c, pl.ds(rbase, RPT)])


@functools.lru_cache(maxsize=1)
def _make_edge_pass():
    return functools.partial(
        pl.kernel,
        mesh=plsc.VectorSubcoreMesh(core_axis_name="c", subcore_axis_name="s"),
        compiler_params=pltpu.CompilerParams(use_tc_tiling_on_sc=False),
        out_type=jax.ShapeDtypeStruct((2, NPAD, DW), jnp.float32),
        scratch_types=[
            pltpu.VMEM_SHARED((NPAD, DW), jnp.float32),
            pltpu.VMEM((3, CHUNK), jnp.int32),
            pltpu.VMEM((3, CHUNK), jnp.int32),
            pltpu.VMEM((3, CHUNK), jnp.int32),
            pltpu.VMEM((2, CHUNK, H), jnp.float32),
            pltpu.VMEM((2, CHUNK, DW), jnp.float32),
            pltpu.SemaphoreType.DMA((2,)),
            pltpu.SemaphoreType.DMA((3,)),
            pltpu.SemaphoreType.DMA((2,)),
        ],
    )(_edge_body)


def _edge_pass(*args):
    return _make_edge_pass()(*args)


def kernel(x, edge_index, emb_W1, emb_b1, emb_bn_g, emb_bn_b, emb_W2, emb_b2,
           gat_W, attn_l, attn_r, gat_bias, bn1_g, bn1_b, ff_W1, ff_b1,
           ff_W2, ff_b2, bn2_g, bn2_b, dec_W1, dec_b1, dec_bn_g, dec_bn_b,
           dec_W2, dec_b2):
    src = edge_index[0]
    dst = edge_index[1]
    npad_e = EPAD - E
    src_p = jnp.concatenate([src, jnp.zeros((npad_e,), jnp.int32)]).reshape(CROWS, CHUNK)
    dst_p = jnp.concatenate([dst, jnp.full((npad_e,), N, jnp.int32)]).reshape(CROWS, CHUNK)
    r = lambda a: a.reshape(1, -1)

    eyeh = jnp.eye(H, dtype=jnp.float32)
    abig = (attn_l[:, :, :, None] * eyeh[None, :, None, :]).reshape(L, D, H)
    bbig = (attn_r[:, :, :, None] * eyeh[None, :, None, :]).reshape(L, D, H)
    e16 = jnp.kron(eyeh, jnp.ones((1, DH), jnp.float32))

    h = pl.pallas_call(_embed_body, out_shape=_f32((N, D)))(
        x, emb_W1, r(emb_b1), r(emb_bn_g), r(emb_bn_b), emb_W2, r(emb_b2))

    for l in range(L):
        hhx, er = pl.pallas_call(
            _attn_body, out_shape=[_f32((2, N, DW)), _f32((N, H))],
        )(h, gat_W[l], abig[l], bbig[l])

        hhx_cat = hhx.reshape(2 * N, DW)
        er_p = jnp.pad(er, ((0, NPAD - N), (0, 0)))
        rstu2 = _edge_pass(src_p, dst_p, hhx_cat, er_p)

        t = pl.pallas_call(
            _post1_body, out_shape=_f32((N, D)),
        )(h, rstu2[0, :N, :128], rstu2[1, :N, :128], rstu2[0, :N, 128:], e16,
          r(gat_bias[l]), r(bn1_g[l]), r(bn1_b[l]))
        h = pl.pallas_call(
            _post2_body, out_shape=_f32((N, D)),
        )(t, ff_W1[l], r(ff_b1[l]), ff_W2[l], r(ff_b2[l]), r(bn2_g[l]), r(bn2_b[l]))

    out = pl.pallas_call(_dec_body, out_shape=_f32((N, DOUT)))(
        h, dec_W1, r(dec_b1), r(dec_bn_g), r(dec_bn_b), dec_W2, r(dec_b2))
    return out
